# Initial kernel scaffold; baseline (speedup 1.0000x reference)
import jax
import jax.numpy as jnp
from jax.experimental import pallas as pl


def kernel(nodes, edges, graph, nodes_res, edges_res, params):
    raise NotImplementedError("write your pallas kernel here")


# speedup vs baseline: 2.7597x; 2.7597x over previous
import functools

import jax
import jax.numpy as jnp
from jax import lax
from jax.experimental import pallas as pl
from jax.experimental.pallas import tpu as pltpu
from jax.experimental.pallas import tpu_sc as plsc

_H = 128
_EPS = 1e-5
_PREC = lax.Precision.HIGHEST


def _ln(x, g, b):
    mu = jnp.mean(x, axis=-1, keepdims=True)
    d = x - mu
    var = jnp.mean(d * d, axis=-1, keepdims=True)
    return d / jnp.sqrt(var + _EPS) * g + b


def _dot(x, w):
    return jnp.dot(x, w, preferred_element_type=jnp.float32, precision=_PREC)


def _edge_pre_body(e_ref, er_ref, wn_ref, we_ref, tn_ref, te_ref):
    x = jnp.concatenate([e_ref[...], er_ref[...]], axis=-1)
    tn_ref[...] = _dot(x, wn_ref[...])
    te_ref[...] = _dot(x, we_ref[...])


def _edge_pre(edges, edges_res, wn, we, be):
    E = edges.shape[0]
    bs_e = pl.BlockSpec((be, _H), lambda i: (i, 0))
    bs_w = pl.BlockSpec((2 * _H, _H), lambda i: (0, 0))
    return pl.pallas_call(
        _edge_pre_body,
        grid=(E // be,),
        in_specs=[bs_e, bs_e, bs_w, bs_w],
        out_specs=[bs_e, bs_e],
        out_shape=(
            jax.ShapeDtypeStruct((E, _H), jnp.float32),
            jax.ShapeDtypeStruct((E, _H), jnp.float32),
        ),
    )(edges, edges_res, wn, we)


def _make_scatter(E, NP):
    mesh = plsc.VectorSubcoreMesh(core_axis_name="c", subcore_axis_name="s")
    nchunks = E // 128
    rows_per_sub = NP // 16
    zrows = 128

    @functools.partial(
        pl.kernel,
        out_type=jax.ShapeDtypeStruct((2, NP, _H), jnp.float32),
        mesh=mesh,
        scratch_types=[
            pltpu.VMEM_SHARED((NP, _H), jnp.float32),
            pltpu.VMEM((zrows, _H), jnp.float32),
        ],
    )
    def k(t_hbm, src_hbm, dst_hbm, out_hbm, acc, zbuf):
        c = lax.axis_index("c")
        s = lax.axis_index("s")

        @pl.loop(0, zrows)
        def _zero_rows(r):
            @pl.loop(0, _H, step=16)
            def _zero_cols(col):
                zbuf[r, pl.ds(col, 16)] = jnp.zeros((16,), jnp.float32)

        @pl.loop(0, rows_per_sub, step=zrows)
        def _fill(r0):
            pltpu.sync_copy(zbuf, acc.at[pl.ds(s * rows_per_sub + r0, zrows)])

        plsc.subcore_barrier()

        def body(t_vmem, s_vmem, d_vmem):
            pltpu.sync_copy(t_vmem, acc.at[s_vmem.at[0]], add=True)
            pltpu.sync_copy(t_vmem, acc.at[d_vmem.at[0]], add=True)

        pltpu.emit_pipeline(
            body,
            grid=(nchunks,),
            in_specs=[
                pl.BlockSpec((128, _H), lambda i: (i, 0)),
                pl.BlockSpec((1, 128), lambda i: (i, 0)),
                pl.BlockSpec((1, 128), lambda i: (i, 0)),
            ],
            core_axis_name=("c", "s"),
            dimension_semantics=(pltpu.PARALLEL,),
        )(t_hbm, src_hbm, dst_hbm)

        plsc.subcore_barrier()
        pltpu.sync_copy(
            acc.at[pl.ds(s * rows_per_sub, rows_per_sub)],
            out_hbm.at[c, pl.ds(s * rows_per_sub, rows_per_sub)],
        )

    return k


def _node_mlp_body(
    n_ref, nr_ref, m0_ref, m1_ref,
    w1_ref, b1_ref, g1_ref, be1_ref,
    w2_ref, b2_ref, g2_ref, be2_ref,
    w3_ref, b3_ref, g3_ref, be3_ref,
    wsrc_ref, wdst_ref,
    nn_ref, a_ref, b_ref,
):
    x = jnp.concatenate([n_ref[...], nr_ref[...]], axis=-1)
    h = _dot(x, w1_ref[...]) + m0_ref[...] + m1_ref[...] + b1_ref[...]
    h = jax.nn.relu(_ln(h, g1_ref[...], be1_ref[...]))
    h = _dot(h, w2_ref[...]) + b2_ref[...]
    h = jax.nn.relu(_ln(h, g2_ref[...], be2_ref[...]))
    h = _dot(h, w3_ref[...]) + b3_ref[...]
    h = jnp.tanh(_ln(h, g3_ref[...], be3_ref[...]))
    nn_ref[...] = h
    a_ref[...] = _dot(h, wsrc_ref[...])
    b_ref[...] = _dot(h, wdst_ref[...])


def _node_mlp(nodes, nodes_res, m0, m1, w1, mats, vecs, wsrc, wdst, bn):
    N = nodes.shape[0]
    bs_n = pl.BlockSpec((bn, _H), lambda i: (i, 0))
    bs_w1 = pl.BlockSpec((2 * _H, _H), lambda i: (0, 0))
    bs_w = pl.BlockSpec((_H, _H), lambda i: (0, 0))
    bs_v = pl.BlockSpec((1, _H), lambda i: (0, 0))
    w2, w3 = mats
    b1, g1, be1, b2, g2, be2, b3, g3, be3 = vecs
    return pl.pallas_call(
        _node_mlp_body,
        grid=(N // bn,),
        in_specs=[bs_n, bs_n, bs_n, bs_n,
                  bs_w1, bs_v, bs_v, bs_v,
                  bs_w, bs_v, bs_v, bs_v,
                  bs_w, bs_v, bs_v, bs_v,
                  bs_w, bs_w],
        out_specs=[bs_n, bs_n, bs_n],
        out_shape=(
            jax.ShapeDtypeStruct((N, _H), jnp.float32),
            jax.ShapeDtypeStruct((N, _H), jnp.float32),
            jax.ShapeDtypeStruct((N, _H), jnp.float32),
        ),
    )(nodes, nodes_res, m0, m1,
      w1, b1, g1, be1, w2, b2, g2, be2, w3, b3, g3, be3, wsrc, wdst)


def _make_gather(E, N):
    mesh = plsc.VectorSubcoreMesh(core_axis_name="c", subcore_axis_name="s")
    nchunks = E // 128

    @functools.partial(
        pl.kernel,
        out_type=(
            jax.ShapeDtypeStruct((E, _H), jnp.float32),
            jax.ShapeDtypeStruct((E, _H), jnp.float32),
        ),
        mesh=mesh,
    )
    def k(a_hbm, b_hbm, src_hbm, dst_hbm, ga_hbm, gb_hbm):
        def body(s_vmem, d_vmem, ga_vmem, gb_vmem):
            pltpu.sync_copy(a_hbm.at[s_vmem.at[0]], ga_vmem)
            pltpu.sync_copy(b_hbm.at[d_vmem.at[0]], gb_vmem)

        pltpu.emit_pipeline(
            body,
            grid=(nchunks,),
            in_specs=[
                pl.BlockSpec((1, 128), lambda i: (i, 0)),
                pl.BlockSpec((1, 128), lambda i: (i, 0)),
            ],
            out_specs=[
                pl.BlockSpec((128, _H), lambda i: (i, 0)),
                pl.BlockSpec((128, _H), lambda i: (i, 0)),
            ],
            core_axis_name=("c", "s"),
            dimension_semantics=(pltpu.PARALLEL,),
        )(src_hbm, dst_hbm, ga_hbm, gb_hbm)

    return k


def _edge_mlp_body(
    ga_ref, gb_ref, te_ref,
    b1_ref, g1_ref, be1_ref,
    w2_ref, b2_ref, g2_ref, be2_ref,
    w3_ref, b3_ref, g3_ref, be3_ref,
    out_ref,
):
    h = ga_ref[...] + gb_ref[...] + te_ref[...] + b1_ref[...]
    h = jax.nn.relu(_ln(h, g1_ref[...], be1_ref[...]))
    h = _dot(h, w2_ref[...]) + b2_ref[...]
    h = jax.nn.relu(_ln(h, g2_ref[...], be2_ref[...]))
    h = _dot(h, w3_ref[...]) + b3_ref[...]
    out_ref[...] = jnp.tanh(_ln(h, g3_ref[...], be3_ref[...]))


def _edge_mlp(ga, gb, te, mats, vecs, be):
    E = ga.shape[0]
    bs_e = pl.BlockSpec((be, _H), lambda i: (i, 0))
    bs_w = pl.BlockSpec((_H, _H), lambda i: (0, 0))
    bs_v = pl.BlockSpec((1, _H), lambda i: (0, 0))
    w2, w3 = mats
    b1, g1, be1, b2, g2, be2, b3, g3, be3 = vecs
    return pl.pallas_call(
        _edge_mlp_body,
        grid=(E // be,),
        in_specs=[bs_e, bs_e, bs_e,
                  bs_v, bs_v, bs_v,
                  bs_w, bs_v, bs_v, bs_v,
                  bs_w, bs_v, bs_v, bs_v],
        out_specs=bs_e,
        out_shape=jax.ShapeDtypeStruct((E, _H), jnp.float32),
    )(ga, gb, te, b1, g1, be1, w2, b2, g2, be2, w3, b3, g3, be3)


def _row(v):
    return v.reshape(1, _H)


def kernel(nodes, edges, graph, nodes_res, edges_res, params):
    N = nodes.shape[0]
    E = edges.shape[0]
    npar = params["node_network"]
    epar = params["edge_network"]
    w1n, w2n, w3n = npar["W"]
    w1e, w2e, w3e = epar["W"]

    t_node, t_edge = _edge_pre(
        edges, edges_res, w1n[2 * _H:], w1e[2 * _H:], be=2000
    )

    src2 = graph[0].reshape(E // 128, 128)
    dst2 = graph[1].reshape(E // 128, 128)

    NP = (N + 127) // 128 * 128
    partials = _make_scatter(E, NP)(t_node, src2, dst2)
    partials = partials[:, :N]

    nvecs = (_row(npar["b"][0]), _row(npar["g"][0]), _row(npar["beta"][0]),
             _row(npar["b"][1]), _row(npar["g"][1]), _row(npar["beta"][1]),
             _row(npar["b"][2]), _row(npar["g"][2]), _row(npar["beta"][2]))
    new_nodes, a, b = _node_mlp(
        nodes, nodes_res, partials[0], partials[1],
        w1n[: 2 * _H], (w2n, w3n), nvecs,
        w1e[:_H], w1e[_H: 2 * _H], bn=2000,
    )

    ga, gb = _make_gather(E, N)(a, b, src2, dst2)

    evecs = (_row(epar["b"][0]), _row(epar["g"][0]), _row(epar["beta"][0]),
             _row(epar["b"][1]), _row(epar["g"][1]), _row(epar["beta"][1]),
             _row(epar["b"][2]), _row(epar["g"][2]), _row(epar["beta"][2]))
    new_edges = _edge_mlp(ga, gb, t_edge, (w2e, w3e), evecs, be=2000)

    return (new_nodes, new_edges)


# speedup vs baseline: 3.2366x; 1.1728x over previous
import functools

import jax
import jax.numpy as jnp
from jax import lax
from jax.experimental import pallas as pl
from jax.experimental.pallas import tpu as pltpu
from jax.experimental.pallas import tpu_sc as plsc

_H = 128
_EPS = 1e-5
_PREC = lax.Precision.HIGHEST


def _ln(x, g, b):
    mu = jnp.mean(x, axis=-1, keepdims=True)
    d = x - mu
    var = jnp.mean(d * d, axis=-1, keepdims=True)
    return d * lax.rsqrt(var + _EPS) * g + b


def _tanh(x):
    return 1.0 - 2.0 / (jnp.exp(2.0 * x) + 1.0)


def _dot(x, w):
    return jnp.dot(x, w, preferred_element_type=jnp.float32, precision=_PREC)


def _edge_pre_body(e_ref, er_ref, w_ref, t_ref):
    x = jnp.concatenate([e_ref[...], er_ref[...]], axis=-1)
    t_ref[...] = _dot(x, w_ref[...])


def _edge_pre(edges, edges_res, w, be):
    E = edges.shape[0]
    bs_e = pl.BlockSpec((be, _H), lambda i: (i, 0))
    bs_w = pl.BlockSpec((2 * _H, _H), lambda i: (0, 0))
    return pl.pallas_call(
        _edge_pre_body,
        grid=(E // be,),
        in_specs=[bs_e, bs_e, bs_w],
        out_specs=bs_e,
        out_shape=jax.ShapeDtypeStruct((E, _H), jnp.float32),
    )(edges, edges_res, w)


def _make_scatter(E, NP):
    mesh = plsc.VectorSubcoreMesh(core_axis_name="c", subcore_axis_name="s")
    nchunks = E // 128
    rows_per_sub = NP // 16
    zrows = 128

    @functools.partial(
        pl.kernel,
        out_type=jax.ShapeDtypeStruct((2, NP, _H), jnp.float32),
        mesh=mesh,
        scratch_types=[
            pltpu.VMEM_SHARED((NP, _H), jnp.float32),
            pltpu.VMEM((zrows, _H), jnp.float32),
        ],
    )
    def k(t_hbm, src_hbm, dst_hbm, out_hbm, acc, zbuf):
        c = lax.axis_index("c")
        s = lax.axis_index("s")

        @pl.loop(0, zrows)
        def _zero_rows(r):
            @pl.loop(0, _H, step=16)
            def _zero_cols(col):
                zbuf[r, pl.ds(col, 16)] = jnp.zeros((16,), jnp.float32)

        @pl.loop(0, rows_per_sub, step=zrows)
        def _fill(r0):
            pltpu.sync_copy(zbuf, acc.at[pl.ds(s * rows_per_sub + r0, zrows)])

        plsc.subcore_barrier()

        def body(t_vmem, s_vmem, d_vmem):
            pltpu.sync_copy(t_vmem, acc.at[s_vmem.at[0]], add=True)
            pltpu.sync_copy(t_vmem, acc.at[d_vmem.at[0]], add=True)

        pltpu.emit_pipeline(
            body,
            grid=(nchunks,),
            in_specs=[
                pl.BlockSpec((128, _H), lambda i: (i, 0)),
                pl.BlockSpec((1, 128), lambda i: (i, 0)),
                pl.BlockSpec((1, 128), lambda i: (i, 0)),
            ],
            core_axis_name=("c", "s"),
            dimension_semantics=(pltpu.PARALLEL,),
        )(t_hbm, src_hbm, dst_hbm)

        plsc.subcore_barrier()
        pltpu.sync_copy(
            acc.at[pl.ds(s * rows_per_sub, rows_per_sub)],
            out_hbm.at[c, pl.ds(s * rows_per_sub, rows_per_sub)],
        )

    return k


def _node_mlp_body(
    n_ref, nr_ref, m0_ref, m1_ref,
    w1_ref, b1_ref, g1_ref, be1_ref,
    w2_ref, b2_ref, g2_ref, be2_ref,
    w3_ref, b3_ref, g3_ref, be3_ref,
    wsrc_ref, wdst_ref,
    nn_ref, a_ref, b_ref,
):
    x = jnp.concatenate([n_ref[...], nr_ref[...]], axis=-1)
    h = _dot(x, w1_ref[...]) + m0_ref[...] + m1_ref[...] + b1_ref[...]
    h = jax.nn.relu(_ln(h, g1_ref[...], be1_ref[...]))
    h = _dot(h, w2_ref[...]) + b2_ref[...]
    h = jax.nn.relu(_ln(h, g2_ref[...], be2_ref[...]))
    h = _dot(h, w3_ref[...]) + b3_ref[...]
    h = _tanh(_ln(h, g3_ref[...], be3_ref[...]))
    nn_ref[...] = h
    a_ref[...] = _dot(h, wsrc_ref[...])
    b_ref[...] = _dot(h, wdst_ref[...])


def _node_mlp(nodes, nodes_res, m0, m1, w1, mats, vecs, wsrc, wdst, bn):
    N = nodes.shape[0]
    bs_n = pl.BlockSpec((bn, _H), lambda i: (i, 0))
    bs_w1 = pl.BlockSpec((2 * _H, _H), lambda i: (0, 0))
    bs_w = pl.BlockSpec((_H, _H), lambda i: (0, 0))
    bs_v = pl.BlockSpec((1, _H), lambda i: (0, 0))
    w2, w3 = mats
    b1, g1, be1, b2, g2, be2, b3, g3, be3 = vecs
    return pl.pallas_call(
        _node_mlp_body,
        grid=(N // bn,),
        in_specs=[bs_n, bs_n, bs_n, bs_n,
                  bs_w1, bs_v, bs_v, bs_v,
                  bs_w, bs_v, bs_v, bs_v,
                  bs_w, bs_v, bs_v, bs_v,
                  bs_w, bs_w],
        out_specs=[bs_n, bs_n, bs_n],
        out_shape=(
            jax.ShapeDtypeStruct((N, _H), jnp.float32),
            jax.ShapeDtypeStruct((N, _H), jnp.float32),
            jax.ShapeDtypeStruct((N, _H), jnp.float32),
        ),
    )(nodes, nodes_res, m0, m1,
      w1, b1, g1, be1, w2, b2, g2, be2, w3, b3, g3, be3, wsrc, wdst)


def _make_gather(E, N):
    mesh = plsc.VectorSubcoreMesh(core_axis_name="c", subcore_axis_name="s")
    nchunks = E // 128

    @functools.partial(
        pl.kernel,
        out_type=(
            jax.ShapeDtypeStruct((E, _H), jnp.float32),
            jax.ShapeDtypeStruct((E, _H), jnp.float32),
        ),
        mesh=mesh,
    )
    def k(a_hbm, b_hbm, src_hbm, dst_hbm, ga_hbm, gb_hbm):
        def body(s_vmem, d_vmem, ga_vmem, gb_vmem):
            pltpu.sync_copy(a_hbm.at[s_vmem.at[0]], ga_vmem)
            pltpu.sync_copy(b_hbm.at[d_vmem.at[0]], gb_vmem)

        pltpu.emit_pipeline(
            body,
            grid=(nchunks,),
            in_specs=[
                pl.BlockSpec((1, 128), lambda i: (i, 0)),
                pl.BlockSpec((1, 128), lambda i: (i, 0)),
            ],
            out_specs=[
                pl.BlockSpec((128, _H), lambda i: (i, 0)),
                pl.BlockSpec((128, _H), lambda i: (i, 0)),
            ],
            core_axis_name=("c", "s"),
            dimension_semantics=(pltpu.PARALLEL,),
        )(src_hbm, dst_hbm, ga_hbm, gb_hbm)

    return k


def _edge_mlp_body(
    ga_ref, gb_ref, te_ref,
    b1_ref, g1_ref, be1_ref,
    w2_ref, b2_ref, g2_ref, be2_ref,
    w3_ref, b3_ref, g3_ref, be3_ref,
    out_ref,
):
    h = ga_ref[...] + gb_ref[...] + te_ref[...] + b1_ref[...]
    h = jax.nn.relu(_ln(h, g1_ref[...], be1_ref[...]))
    h = _dot(h, w2_ref[...]) + b2_ref[...]
    h = jax.nn.relu(_ln(h, g2_ref[...], be2_ref[...]))
    h = _dot(h, w3_ref[...]) + b3_ref[...]
    out_ref[...] = _tanh(_ln(h, g3_ref[...], be3_ref[...]))


def _edge_mlp(ga, gb, te, mats, vecs, be):
    E = ga.shape[0]
    bs_e = pl.BlockSpec((be, _H), lambda i: (i, 0))
    bs_w = pl.BlockSpec((_H, _H), lambda i: (0, 0))
    bs_v = pl.BlockSpec((1, _H), lambda i: (0, 0))
    w2, w3 = mats
    b1, g1, be1, b2, g2, be2, b3, g3, be3 = vecs
    return pl.pallas_call(
        _edge_mlp_body,
        grid=(E // be,),
        in_specs=[bs_e, bs_e, bs_e,
                  bs_v, bs_v, bs_v,
                  bs_w, bs_v, bs_v, bs_v,
                  bs_w, bs_v, bs_v, bs_v],
        out_specs=bs_e,
        out_shape=jax.ShapeDtypeStruct((E, _H), jnp.float32),
    )(ga, gb, te, b1, g1, be1, w2, b2, g2, be2, w3, b3, g3, be3)


def _row(v):
    return v.reshape(1, _H)


def kernel(nodes, edges, graph, nodes_res, edges_res, params):
    N = nodes.shape[0]
    E = edges.shape[0]
    npar = params["node_network"]
    epar = params["edge_network"]
    w1n, w2n, w3n = npar["W"]
    w1e, w2e, w3e = epar["W"]

    t_node = _edge_pre(edges, edges_res, w1n[2 * _H:], be=4000)
    t_edge = _edge_pre(edges, edges_res, w1e[2 * _H:], be=4000)

    src2 = graph[0].reshape(E // 128, 128)
    dst2 = graph[1].reshape(E // 128, 128)

    NP = (N + 127) // 128 * 128
    partials = _make_scatter(E, NP)(t_node, src2, dst2)
    partials = partials[:, :N]

    nvecs = (_row(npar["b"][0]), _row(npar["g"][0]), _row(npar["beta"][0]),
             _row(npar["b"][1]), _row(npar["g"][1]), _row(npar["beta"][1]),
             _row(npar["b"][2]), _row(npar["g"][2]), _row(npar["beta"][2]))
    new_nodes, a, b = _node_mlp(
        nodes, nodes_res, partials[0], partials[1],
        w1n[: 2 * _H], (w2n, w3n), nvecs,
        w1e[:_H], w1e[_H: 2 * _H], bn=2000,
    )

    ga, gb = _make_gather(E, N)(a, b, src2, dst2)

    evecs = (_row(epar["b"][0]), _row(epar["g"][0]), _row(epar["beta"][0]),
             _row(epar["b"][1]), _row(epar["g"][1]), _row(epar["beta"][1]),
             _row(epar["b"][2]), _row(epar["g"][2]), _row(epar["beta"][2]))
    new_edges = _edge_mlp(ga, gb, t_edge, (w2e, w3e), evecs, be=4000)

    return (new_nodes, new_edges)


# speedup vs baseline: 5.0409x; 1.5575x over previous
import functools

import jax
import jax.numpy as jnp
from jax import lax
from jax.experimental import pallas as pl
from jax.experimental.pallas import tpu as pltpu
from jax.experimental.pallas import tpu_sc as plsc

_H = 128
_EPS = 1e-5
_PREC = None


def _ln(x, g, b):
    mu = jnp.mean(x, axis=-1, keepdims=True)
    d = x - mu
    var = jnp.mean(d * d, axis=-1, keepdims=True)
    return d * lax.rsqrt(var + _EPS) * g + b


def _tanh(x):
    return 1.0 - 2.0 / (jnp.exp(2.0 * x) + 1.0)


def _dot(x, w):
    return jnp.dot(x, w, preferred_element_type=jnp.float32, precision=_PREC)


def _edge_pre_body(e_ref, er_ref, wn_ref, we_ref, tn_ref, te_ref):
    x = jnp.concatenate([e_ref[...], er_ref[...]], axis=-1)
    tn_ref[...] = _dot(x, wn_ref[...])
    te_ref[...] = _dot(x, we_ref[...])


def _edge_pre(edges, edges_res, wn, we, be):
    E = edges.shape[0]
    bs_e = pl.BlockSpec((be, _H), lambda i: (i, 0))
    bs_w = pl.BlockSpec((2 * _H, _H), lambda i: (0, 0))
    return pl.pallas_call(
        _edge_pre_body,
        grid=(E // be,),
        in_specs=[bs_e, bs_e, bs_w, bs_w],
        out_specs=[bs_e, bs_e],
        out_shape=(
            jax.ShapeDtypeStruct((E, _H), jnp.float32),
            jax.ShapeDtypeStruct((E, _H), jnp.float32),
        ),
    )(edges, edges_res, wn, we)


def _make_scatter(E, NP):
    mesh = plsc.VectorSubcoreMesh(core_axis_name="c", subcore_axis_name="s")
    nchunks = E // 128
    rows_per_sub = NP // 16
    zrows = 128

    @functools.partial(
        pl.kernel,
        out_type=jax.ShapeDtypeStruct((2, NP, _H), jnp.float32),
        mesh=mesh,
        scratch_types=[
            pltpu.VMEM_SHARED((NP, _H), jnp.float32),
            pltpu.VMEM((zrows, _H), jnp.float32),
        ],
    )
    def k(t_hbm, src_hbm, dst_hbm, out_hbm, acc, zbuf):
        c = lax.axis_index("c")
        s = lax.axis_index("s")

        @pl.loop(0, zrows)
        def _zero_rows(r):
            @pl.loop(0, _H, step=16)
            def _zero_cols(col):
                zbuf[r, pl.ds(col, 16)] = jnp.zeros((16,), jnp.float32)

        @pl.loop(0, rows_per_sub, step=zrows)
        def _fill(r0):
            pltpu.sync_copy(zbuf, acc.at[pl.ds(s * rows_per_sub + r0, zrows)])

        plsc.subcore_barrier()

        def body(t_vmem, s_vmem, d_vmem):
            pltpu.sync_copy(t_vmem, acc.at[s_vmem.at[0]], add=True)
            pltpu.sync_copy(t_vmem, acc.at[d_vmem.at[0]], add=True)

        pltpu.emit_pipeline(
            body,
            grid=(nchunks,),
            in_specs=[
                pl.BlockSpec((128, _H), lambda i: (i, 0)),
                pl.BlockSpec((1, 128), lambda i: (i, 0)),
                pl.BlockSpec((1, 128), lambda i: (i, 0)),
            ],
            core_axis_name=("c", "s"),
            dimension_semantics=(pltpu.PARALLEL,),
        )(t_hbm, src_hbm, dst_hbm)

        plsc.subcore_barrier()
        pltpu.sync_copy(
            acc.at[pl.ds(s * rows_per_sub, rows_per_sub)],
            out_hbm.at[c, pl.ds(s * rows_per_sub, rows_per_sub)],
        )

    return k


def _node_mlp_body(
    n_ref, nr_ref, m0_ref, m1_ref,
    w1_ref, b1_ref, g1_ref, be1_ref,
    w2_ref, b2_ref, g2_ref, be2_ref,
    w3_ref, b3_ref, g3_ref, be3_ref,
    wsrc_ref, wdst_ref,
    nn_ref, a_ref, b_ref,
):
    x = jnp.concatenate([n_ref[...], nr_ref[...]], axis=-1)
    h = _dot(x, w1_ref[...]) + m0_ref[...] + m1_ref[...] + b1_ref[...]
    h = jax.nn.relu(_ln(h, g1_ref[...], be1_ref[...]))
    h = _dot(h, w2_ref[...]) + b2_ref[...]
    h = jax.nn.relu(_ln(h, g2_ref[...], be2_ref[...]))
    h = _dot(h, w3_ref[...]) + b3_ref[...]
    h = _tanh(_ln(h, g3_ref[...], be3_ref[...]))
    nn_ref[...] = h
    a_ref[...] = _dot(h, wsrc_ref[...])
    b_ref[...] = _dot(h, wdst_ref[...])


def _node_mlp(nodes, nodes_res, m0, m1, w1, mats, vecs, wsrc, wdst, bn):
    N = nodes.shape[0]
    bs_n = pl.BlockSpec((bn, _H), lambda i: (i, 0))
    bs_w1 = pl.BlockSpec((2 * _H, _H), lambda i: (0, 0))
    bs_w = pl.BlockSpec((_H, _H), lambda i: (0, 0))
    bs_v = pl.BlockSpec((1, _H), lambda i: (0, 0))
    w2, w3 = mats
    b1, g1, be1, b2, g2, be2, b3, g3, be3 = vecs
    return pl.pallas_call(
        _node_mlp_body,
        grid=(N // bn,),
        in_specs=[bs_n, bs_n, bs_n, bs_n,
                  bs_w1, bs_v, bs_v, bs_v,
                  bs_w, bs_v, bs_v, bs_v,
                  bs_w, bs_v, bs_v, bs_v,
                  bs_w, bs_w],
        out_specs=[bs_n, bs_n, bs_n],
        out_shape=(
            jax.ShapeDtypeStruct((N, _H), jnp.float32),
            jax.ShapeDtypeStruct((N, _H), jnp.float32),
            jax.ShapeDtypeStruct((N, _H), jnp.float32),
        ),
    )(nodes, nodes_res, m0, m1,
      w1, b1, g1, be1, w2, b2, g2, be2, w3, b3, g3, be3, wsrc, wdst)


def _make_gather(E, N):
    mesh = plsc.VectorSubcoreMesh(core_axis_name="c", subcore_axis_name="s")
    nchunks = E // 128

    @functools.partial(
        pl.kernel,
        out_type=(
            jax.ShapeDtypeStruct((E, _H), jnp.float32),
            jax.ShapeDtypeStruct((E, _H), jnp.float32),
        ),
        mesh=mesh,
    )
    def k(a_hbm, b_hbm, src_hbm, dst_hbm, ga_hbm, gb_hbm):
        def body(s_vmem, d_vmem, ga_vmem, gb_vmem):
            pltpu.sync_copy(a_hbm.at[s_vmem.at[0]], ga_vmem)
            pltpu.sync_copy(b_hbm.at[d_vmem.at[0]], gb_vmem)

        pltpu.emit_pipeline(
            body,
            grid=(nchunks,),
            in_specs=[
                pl.BlockSpec((1, 128), lambda i: (i, 0)),
                pl.BlockSpec((1, 128), lambda i: (i, 0)),
            ],
            out_specs=[
                pl.BlockSpec((128, _H), lambda i: (i, 0)),
                pl.BlockSpec((128, _H), lambda i: (i, 0)),
            ],
            core_axis_name=("c", "s"),
            dimension_semantics=(pltpu.PARALLEL,),
        )(src_hbm, dst_hbm, ga_hbm, gb_hbm)

    return k


def _edge_mlp_body(
    ga_ref, gb_ref, te_ref,
    b1_ref, g1_ref, be1_ref,
    w2_ref, b2_ref, g2_ref, be2_ref,
    w3_ref, b3_ref, g3_ref, be3_ref,
    out_ref,
):
    h = ga_ref[...] + gb_ref[...] + te_ref[...] + b1_ref[...]
    h = jax.nn.relu(_ln(h, g1_ref[...], be1_ref[...]))
    h = _dot(h, w2_ref[...]) + b2_ref[...]
    h = jax.nn.relu(_ln(h, g2_ref[...], be2_ref[...]))
    h = _dot(h, w3_ref[...]) + b3_ref[...]
    out_ref[...] = _tanh(_ln(h, g3_ref[...], be3_ref[...]))


def _edge_mlp(ga, gb, te, mats, vecs, be):
    E = ga.shape[0]
    bs_e = pl.BlockSpec((be, _H), lambda i: (i, 0))
    bs_w = pl.BlockSpec((_H, _H), lambda i: (0, 0))
    bs_v = pl.BlockSpec((1, _H), lambda i: (0, 0))
    w2, w3 = mats
    b1, g1, be1, b2, g2, be2, b3, g3, be3 = vecs
    return pl.pallas_call(
        _edge_mlp_body,
        grid=(E // be,),
        in_specs=[bs_e, bs_e, bs_e,
                  bs_v, bs_v, bs_v,
                  bs_w, bs_v, bs_v, bs_v,
                  bs_w, bs_v, bs_v, bs_v],
        out_specs=bs_e,
        out_shape=jax.ShapeDtypeStruct((E, _H), jnp.float32),
    )(ga, gb, te, b1, g1, be1, w2, b2, g2, be2, w3, b3, g3, be3)


def _row(v):
    return v.reshape(1, _H)


def kernel(nodes, edges, graph, nodes_res, edges_res, params):
    N = nodes.shape[0]
    E = edges.shape[0]
    npar = params["node_network"]
    epar = params["edge_network"]
    w1n, w2n, w3n = npar["W"]
    w1e, w2e, w3e = epar["W"]

    t_node, t_edge = _edge_pre(
        edges, edges_res, w1n[2 * _H:], w1e[2 * _H:], be=8000
    )

    src2 = graph[0].reshape(E // 128, 128)
    dst2 = graph[1].reshape(E // 128, 128)

    NP = (N + 127) // 128 * 128
    partials = _make_scatter(E, NP)(t_node, src2, dst2)
    partials = partials[:, :N]

    nvecs = (_row(npar["b"][0]), _row(npar["g"][0]), _row(npar["beta"][0]),
             _row(npar["b"][1]), _row(npar["g"][1]), _row(npar["beta"][1]),
             _row(npar["b"][2]), _row(npar["g"][2]), _row(npar["beta"][2]))
    new_nodes, a, b = _node_mlp(
        nodes, nodes_res, partials[0], partials[1],
        w1n[: 2 * _H], (w2n, w3n), nvecs,
        w1e[:_H], w1e[_H: 2 * _H], bn=2000,
    )

    ga, gb = _make_gather(E, N)(a, b, src2, dst2)

    evecs = (_row(epar["b"][0]), _row(epar["g"][0]), _row(epar["beta"][0]),
             _row(epar["b"][1]), _row(epar["g"][1]), _row(epar["beta"][1]),
             _row(epar["b"][2]), _row(epar["g"][2]), _row(epar["beta"][2]))
    new_edges = _edge_mlp(ga, gb, t_edge, (w2e, w3e), evecs, be=8000)

    return (new_nodes, new_edges)


# speedup vs baseline: 5.1251x; 1.0167x over previous
import functools

import jax
import jax.numpy as jnp
from jax import lax
from jax.experimental import pallas as pl
from jax.experimental.pallas import tpu as pltpu
from jax.experimental.pallas import tpu_sc as plsc

_H = 128
_EPS = 1e-5
_PREC = None


def _ln(x, g, b):
    mu = jnp.mean(x, axis=-1, keepdims=True)
    d = x - mu
    var = jnp.mean(d * d, axis=-1, keepdims=True)
    return d * lax.rsqrt(var + _EPS) * g + b


def _tanh(x):
    return 1.0 - 2.0 / (jnp.exp(2.0 * x) + 1.0)


def _dot(x, w):
    return jnp.dot(x, w, preferred_element_type=jnp.float32, precision=_PREC)


def _edge_pre_body(e_ref, er_ref, wn_ref, we_ref, tn_ref, te_ref):
    x = jnp.concatenate([e_ref[...], er_ref[...]], axis=-1)
    tn_ref[...] = _dot(x, wn_ref[...])
    te_ref[...] = _dot(x, we_ref[...]).astype(jnp.bfloat16)


def _edge_pre(edges, edges_res, wn, we, be):
    E = edges.shape[0]
    bs_e = pl.BlockSpec((be, _H), lambda i: (i, 0))
    bs_w = pl.BlockSpec((2 * _H, _H), lambda i: (0, 0))
    return pl.pallas_call(
        _edge_pre_body,
        grid=(E // be,),
        in_specs=[bs_e, bs_e, bs_w, bs_w],
        out_specs=[bs_e, bs_e],
        out_shape=(
            jax.ShapeDtypeStruct((E, _H), jnp.float32),
            jax.ShapeDtypeStruct((E, _H), jnp.bfloat16),
        ),
    )(edges, edges_res, wn, we)


def _make_scatter(E, NP):
    mesh = plsc.VectorSubcoreMesh(core_axis_name="c", subcore_axis_name="s")
    nchunks = E // 128
    rows_per_sub = NP // 16
    zrows = 128

    @functools.partial(
        pl.kernel,
        out_type=jax.ShapeDtypeStruct((2, NP, _H), jnp.float32),
        mesh=mesh,
        scratch_types=[
            pltpu.VMEM_SHARED((NP, _H), jnp.float32),
            pltpu.VMEM((zrows, _H), jnp.float32),
        ],
    )
    def k(t_hbm, src_hbm, dst_hbm, out_hbm, acc, zbuf):
        c = lax.axis_index("c")
        s = lax.axis_index("s")

        @pl.loop(0, zrows)
        def _zero_rows(r):
            @pl.loop(0, _H, step=16)
            def _zero_cols(col):
                zbuf[r, pl.ds(col, 16)] = jnp.zeros((16,), jnp.float32)

        @pl.loop(0, rows_per_sub, step=zrows)
        def _fill(r0):
            pltpu.sync_copy(zbuf, acc.at[pl.ds(s * rows_per_sub + r0, zrows)])

        plsc.subcore_barrier()

        def body(t_vmem, s_vmem, d_vmem):
            pltpu.sync_copy(t_vmem, acc.at[s_vmem.at[0]], add=True)
            pltpu.sync_copy(t_vmem, acc.at[d_vmem.at[0]], add=True)

        pltpu.emit_pipeline(
            body,
            grid=(nchunks,),
            in_specs=[
                pl.BlockSpec((128, _H), lambda i: (i, 0)),
                pl.BlockSpec((1, 128), lambda i: (i, 0)),
                pl.BlockSpec((1, 128), lambda i: (i, 0)),
            ],
            core_axis_name=("c", "s"),
            dimension_semantics=(pltpu.PARALLEL,),
        )(t_hbm, src_hbm, dst_hbm)

        plsc.subcore_barrier()
        pltpu.sync_copy(
            acc.at[pl.ds(s * rows_per_sub, rows_per_sub)],
            out_hbm.at[c, pl.ds(s * rows_per_sub, rows_per_sub)],
        )

    return k


def _node_mlp_body(
    n_ref, nr_ref, m0_ref, m1_ref,
    w1_ref, b1_ref, g1_ref, be1_ref,
    w2_ref, b2_ref, g2_ref, be2_ref,
    w3_ref, b3_ref, g3_ref, be3_ref,
    wsrc_ref, wdst_ref,
    nn_ref, a_ref, b_ref,
):
    x = jnp.concatenate([n_ref[...], nr_ref[...]], axis=-1)
    h = _dot(x, w1_ref[...]) + m0_ref[...] + m1_ref[...] + b1_ref[...]
    h = jax.nn.relu(_ln(h, g1_ref[...], be1_ref[...]))
    h = _dot(h, w2_ref[...]) + b2_ref[...]
    h = jax.nn.relu(_ln(h, g2_ref[...], be2_ref[...]))
    h = _dot(h, w3_ref[...]) + b3_ref[...]
    h = _tanh(_ln(h, g3_ref[...], be3_ref[...]))
    nn_ref[...] = h
    a_ref[...] = _dot(h, wsrc_ref[...])
    b_ref[...] = _dot(h, wdst_ref[...])


def _node_mlp(nodes, nodes_res, m0, m1, w1, mats, vecs, wsrc, wdst, bn):
    N = nodes.shape[0]
    bs_n = pl.BlockSpec((bn, _H), lambda i: (i, 0))
    bs_w1 = pl.BlockSpec((2 * _H, _H), lambda i: (0, 0))
    bs_w = pl.BlockSpec((_H, _H), lambda i: (0, 0))
    bs_v = pl.BlockSpec((1, _H), lambda i: (0, 0))
    w2, w3 = mats
    b1, g1, be1, b2, g2, be2, b3, g3, be3 = vecs
    return pl.pallas_call(
        _node_mlp_body,
        grid=(N // bn,),
        in_specs=[bs_n, bs_n, bs_n, bs_n,
                  bs_w1, bs_v, bs_v, bs_v,
                  bs_w, bs_v, bs_v, bs_v,
                  bs_w, bs_v, bs_v, bs_v,
                  bs_w, bs_w],
        out_specs=[bs_n, bs_n, bs_n],
        out_shape=(
            jax.ShapeDtypeStruct((N, _H), jnp.float32),
            jax.ShapeDtypeStruct((N, _H), jnp.float32),
            jax.ShapeDtypeStruct((N, _H), jnp.float32),
        ),
    )(nodes, nodes_res, m0, m1,
      w1, b1, g1, be1, w2, b2, g2, be2, w3, b3, g3, be3, wsrc, wdst)


def _make_gather(E, N):
    mesh = plsc.VectorSubcoreMesh(core_axis_name="c", subcore_axis_name="s")
    nchunks = E // 128

    @functools.partial(
        pl.kernel,
        out_type=(
            jax.ShapeDtypeStruct((E, _H), jnp.float32),
            jax.ShapeDtypeStruct((E, _H), jnp.float32),
        ),
        mesh=mesh,
    )
    def k(a_hbm, b_hbm, src_hbm, dst_hbm, ga_hbm, gb_hbm):
        def body(s_vmem, d_vmem, ga_vmem, gb_vmem):
            pltpu.sync_copy(a_hbm.at[s_vmem.at[0]], ga_vmem)
            pltpu.sync_copy(b_hbm.at[d_vmem.at[0]], gb_vmem)

        pltpu.emit_pipeline(
            body,
            grid=(nchunks,),
            in_specs=[
                pl.BlockSpec((1, 128), lambda i: (i, 0)),
                pl.BlockSpec((1, 128), lambda i: (i, 0)),
            ],
            out_specs=[
                pl.BlockSpec((128, _H), lambda i: (i, 0)),
                pl.BlockSpec((128, _H), lambda i: (i, 0)),
            ],
            core_axis_name=("c", "s"),
            dimension_semantics=(pltpu.PARALLEL,),
        )(src_hbm, dst_hbm, ga_hbm, gb_hbm)

    return k


def _edge_mlp_body(
    ga_ref, gb_ref, te_ref,
    b1_ref, g1_ref, be1_ref,
    w2_ref, b2_ref, g2_ref, be2_ref,
    w3_ref, b3_ref, g3_ref, be3_ref,
    out_ref,
):
    h = (ga_ref[...].astype(jnp.float32) + gb_ref[...].astype(jnp.float32)
         + te_ref[...].astype(jnp.float32) + b1_ref[...])
    h = jax.nn.relu(_ln(h, g1_ref[...], be1_ref[...]))
    h = _dot(h, w2_ref[...]) + b2_ref[...]
    h = jax.nn.relu(_ln(h, g2_ref[...], be2_ref[...]))
    h = _dot(h, w3_ref[...]) + b3_ref[...]
    out_ref[...] = _tanh(_ln(h, g3_ref[...], be3_ref[...]))


def _edge_mlp(ga, gb, te, te_block0, mats, vecs, be):
    Ec = ga.shape[0]
    bs_e = pl.BlockSpec((be, _H), lambda i: (i, 0))
    bs_te = pl.BlockSpec((be, _H), lambda i: (te_block0 + i, 0))
    bs_w = pl.BlockSpec((_H, _H), lambda i: (0, 0))
    bs_v = pl.BlockSpec((1, _H), lambda i: (0, 0))
    w2, w3 = mats
    b1, g1, be1, b2, g2, be2, b3, g3, be3 = vecs
    return pl.pallas_call(
        _edge_mlp_body,
        grid=(Ec // be,),
        in_specs=[bs_e, bs_e, bs_te,
                  bs_v, bs_v, bs_v,
                  bs_w, bs_v, bs_v, bs_v,
                  bs_w, bs_v, bs_v, bs_v],
        out_specs=bs_e,
        out_shape=jax.ShapeDtypeStruct((Ec, _H), jnp.float32),
    )(ga, gb, te, b1, g1, be1, w2, b2, g2, be2, w3, b3, g3, be3)


def _row(v):
    return v.reshape(1, _H)


def kernel(nodes, edges, graph, nodes_res, edges_res, params):
    N = nodes.shape[0]
    E = edges.shape[0]
    npar = params["node_network"]
    epar = params["edge_network"]
    w1n, w2n, w3n = npar["W"]
    w1e, w2e, w3e = epar["W"]

    t_node, t_edge = _edge_pre(
        edges, edges_res, w1n[2 * _H:], w1e[2 * _H:], be=8000
    )

    src2 = graph[0].reshape(E // 128, 128)
    dst2 = graph[1].reshape(E // 128, 128)

    NP = (N + 127) // 128 * 128
    partials = _make_scatter(E, NP)(t_node, src2, dst2)
    partials = partials[:, :N]

    nvecs = (_row(npar["b"][0]), _row(npar["g"][0]), _row(npar["beta"][0]),
             _row(npar["b"][1]), _row(npar["g"][1]), _row(npar["beta"][1]),
             _row(npar["b"][2]), _row(npar["g"][2]), _row(npar["beta"][2]))
    new_nodes, a, b = _node_mlp(
        nodes, nodes_res, partials[0], partials[1],
        w1n[: 2 * _H], (w2n, w3n), nvecs,
        w1e[:_H], w1e[_H: 2 * _H], bn=2000,
    )

    evecs = (_row(epar["b"][0]), _row(epar["g"][0]), _row(epar["beta"][0]),
             _row(epar["b"][1]), _row(epar["g"][1]), _row(epar["beta"][1]),
             _row(epar["b"][2]), _row(epar["g"][2]), _row(epar["beta"][2]))
    CH = 4
    BE = 8000
    Ec = E // CH
    rows_c = (E // 128) // CH
    gather_k = _make_gather(Ec, N)
    gs = []
    for i in range(CH):
        s2 = lax.slice_in_dim(src2, i * rows_c, (i + 1) * rows_c)
        d2 = lax.slice_in_dim(dst2, i * rows_c, (i + 1) * rows_c)
        gs.append(gather_k(a, b, s2, d2))
    parts = []
    for i in range(CH):
        ga, gb = gs[i]
        parts.append(
            _edge_mlp(ga, gb, t_edge, i * (Ec // BE), (w2e, w3e), evecs,
                      be=BE)
        )
    new_edges = jnp.concatenate(parts, axis=0)

    return (new_nodes, new_edges)


# speedup vs baseline: 5.8499x; 1.1414x over previous
import functools

import jax
import jax.numpy as jnp
from jax import lax
from jax.experimental import pallas as pl
from jax.experimental.pallas import tpu as pltpu
from jax.experimental.pallas import tpu_sc as plsc

_H = 128
_EPS = 1e-5
_PREC = None


def _ln(x, g, b):
    mu = jnp.mean(x, axis=-1, keepdims=True)
    d = x - mu
    var = jnp.mean(d * d, axis=-1, keepdims=True)
    return d * lax.rsqrt(var + _EPS) * g + b


def _tanh(x):
    return 1.0 - 2.0 / (jnp.exp(2.0 * x) + 1.0)


def _dot(x, w):
    return jnp.dot(x, w, preferred_element_type=jnp.float32, precision=_PREC)


def _edge_pre_body(e_ref, er_ref, wn_ref, we_ref, tn_ref, te_ref):
    x = jnp.concatenate([e_ref[...], er_ref[...]], axis=-1)
    tn_ref[...] = _dot(x, wn_ref[...])
    te_ref[...] = _dot(x, we_ref[...]).astype(jnp.bfloat16)


def _edge_pre(edges, edges_res, wn, we, be):
    E = edges.shape[0]
    bs_e = pl.BlockSpec((be, _H), lambda i: (i, 0))
    bs_w = pl.BlockSpec((2 * _H, _H), lambda i: (0, 0))
    return pl.pallas_call(
        _edge_pre_body,
        grid=(E // be,),
        in_specs=[bs_e, bs_e, bs_w, bs_w],
        out_specs=[bs_e, bs_e],
        out_shape=(
            jax.ShapeDtypeStruct((E, _H), jnp.float32),
            jax.ShapeDtypeStruct((E, _H), jnp.bfloat16),
        ),
    )(edges, edges_res, wn, we)


def _make_scatter(E, NP):
    mesh = plsc.VectorSubcoreMesh(core_axis_name="c", subcore_axis_name="s")
    nchunks = E // 128
    rows_per_sub = NP // 16
    zrows = 128

    @functools.partial(
        pl.kernel,
        out_type=jax.ShapeDtypeStruct((2, NP, _H), jnp.float32),
        mesh=mesh,
        scratch_types=[
            pltpu.VMEM_SHARED((NP, _H), jnp.float32),
            pltpu.VMEM((zrows, _H), jnp.float32),
            pltpu.SemaphoreType.DMA,
            pltpu.SemaphoreType.DMA,
        ],
    )
    def k(t_hbm, src_hbm, dst_hbm, out_hbm, acc, zbuf, sem1, sem2):
        c = lax.axis_index("c")
        s = lax.axis_index("s")

        @pl.loop(0, zrows)
        def _zero_rows(r):
            @pl.loop(0, _H, step=16)
            def _zero_cols(col):
                zbuf[r, pl.ds(col, 16)] = jnp.zeros((16,), jnp.float32)

        @pl.loop(0, rows_per_sub, step=zrows)
        def _fill(r0):
            pltpu.sync_copy(zbuf, acc.at[pl.ds(s * rows_per_sub + r0, zrows)])

        plsc.subcore_barrier()

        def body(t_vmem, s_vmem, d_vmem):
            c1 = pltpu.async_copy(t_vmem, acc.at[s_vmem.at[0]], sem1, add=True)
            c2 = pltpu.async_copy(t_vmem, acc.at[d_vmem.at[0]], sem2, add=True)
            c1.wait()
            c2.wait()

        pltpu.emit_pipeline(
            body,
            grid=(nchunks,),
            in_specs=[
                pl.BlockSpec((128, _H), lambda i: (i, 0)),
                pl.BlockSpec((1, 128), lambda i: (i, 0)),
                pl.BlockSpec((1, 128), lambda i: (i, 0)),
            ],
            core_axis_name=("c", "s"),
            dimension_semantics=(pltpu.PARALLEL,),
        )(t_hbm, src_hbm, dst_hbm)

        plsc.subcore_barrier()
        pltpu.sync_copy(
            acc.at[pl.ds(s * rows_per_sub, rows_per_sub)],
            out_hbm.at[c, pl.ds(s * rows_per_sub, rows_per_sub)],
        )

    return k


def _node_mlp_body(
    n_ref, nr_ref, m0_ref, m1_ref,
    w1_ref, b1_ref, g1_ref, be1_ref,
    w2_ref, b2_ref, g2_ref, be2_ref,
    w3_ref, b3_ref, g3_ref, be3_ref,
    wsrc_ref, wdst_ref,
    nn_ref, a_ref, b_ref,
):
    x = jnp.concatenate([n_ref[...], nr_ref[...]], axis=-1)
    h = _dot(x, w1_ref[...]) + m0_ref[...] + m1_ref[...] + b1_ref[...]
    h = jax.nn.relu(_ln(h, g1_ref[...], be1_ref[...]))
    h = _dot(h, w2_ref[...]) + b2_ref[...]
    h = jax.nn.relu(_ln(h, g2_ref[...], be2_ref[...]))
    h = _dot(h, w3_ref[...]) + b3_ref[...]
    h = _tanh(_ln(h, g3_ref[...], be3_ref[...]))
    nn_ref[...] = h
    a_ref[...] = _dot(h, wsrc_ref[...])
    b_ref[...] = _dot(h, wdst_ref[...])


def _node_mlp(nodes, nodes_res, m0, m1, w1, mats, vecs, wsrc, wdst, bn):
    N = nodes.shape[0]
    bs_n = pl.BlockSpec((bn, _H), lambda i: (i, 0))
    bs_w1 = pl.BlockSpec((2 * _H, _H), lambda i: (0, 0))
    bs_w = pl.BlockSpec((_H, _H), lambda i: (0, 0))
    bs_v = pl.BlockSpec((1, _H), lambda i: (0, 0))
    w2, w3 = mats
    b1, g1, be1, b2, g2, be2, b3, g3, be3 = vecs
    return pl.pallas_call(
        _node_mlp_body,
        grid=(N // bn,),
        in_specs=[bs_n, bs_n, bs_n, bs_n,
                  bs_w1, bs_v, bs_v, bs_v,
                  bs_w, bs_v, bs_v, bs_v,
                  bs_w, bs_v, bs_v, bs_v,
                  bs_w, bs_w],
        out_specs=[bs_n, bs_n, bs_n],
        out_shape=(
            jax.ShapeDtypeStruct((N, _H), jnp.float32),
            jax.ShapeDtypeStruct((N, _H), jnp.float32),
            jax.ShapeDtypeStruct((N, _H), jnp.float32),
        ),
    )(nodes, nodes_res, m0, m1,
      w1, b1, g1, be1, w2, b2, g2, be2, w3, b3, g3, be3, wsrc, wdst)


def _make_gather(E, N):
    mesh = plsc.VectorSubcoreMesh(core_axis_name="c", subcore_axis_name="s")
    nchunks = E // 128

    @functools.partial(
        pl.kernel,
        out_type=(
            jax.ShapeDtypeStruct((E, _H), jnp.float32),
            jax.ShapeDtypeStruct((E, _H), jnp.float32),
        ),
        mesh=mesh,
    )
    def k(a_hbm, b_hbm, src_hbm, dst_hbm, ga_hbm, gb_hbm):
        def body(s_vmem, d_vmem, ga_vmem, gb_vmem):
            pltpu.sync_copy(a_hbm.at[s_vmem.at[0]], ga_vmem)
            pltpu.sync_copy(b_hbm.at[d_vmem.at[0]], gb_vmem)

        pltpu.emit_pipeline(
            body,
            grid=(nchunks,),
            in_specs=[
                pl.BlockSpec((1, 128), lambda i: (i, 0)),
                pl.BlockSpec((1, 128), lambda i: (i, 0)),
            ],
            out_specs=[
                pl.BlockSpec((128, _H), lambda i: (i, 0)),
                pl.BlockSpec((128, _H), lambda i: (i, 0)),
            ],
            core_axis_name=("c", "s"),
            dimension_semantics=(pltpu.PARALLEL,),
        )(src_hbm, dst_hbm, ga_hbm, gb_hbm)

    return k


def _edge_mlp_body(
    buf_ref, ga_ref, gb_ref, te_ref,
    b1_ref, g1_ref, be1_ref,
    w2_ref, b2_ref, g2_ref, be2_ref,
    w3_ref, b3_ref, g3_ref, be3_ref,
    out_ref,
):
    h = (ga_ref[...].astype(jnp.float32) + gb_ref[...].astype(jnp.float32)
         + te_ref[...].astype(jnp.float32) + b1_ref[...])
    h = jax.nn.relu(_ln(h, g1_ref[...], be1_ref[...]))
    h = _dot(h, w2_ref[...]) + b2_ref[...]
    h = jax.nn.relu(_ln(h, g2_ref[...], be2_ref[...]))
    h = _dot(h, w3_ref[...]) + b3_ref[...]
    out_ref[...] = _tanh(_ln(h, g3_ref[...], be3_ref[...]))


def _edge_mlp(buf, ga, gb, te, block0, mats, vecs, be):
    E = buf.shape[0]
    Ec = ga.shape[0]
    bs_e = pl.BlockSpec((be, _H), lambda i: (i, 0))
    bs_full = pl.BlockSpec((be, _H), lambda i: (block0 + i, 0))
    bs_w = pl.BlockSpec((_H, _H), lambda i: (0, 0))
    bs_v = pl.BlockSpec((1, _H), lambda i: (0, 0))
    w2, w3 = mats
    b1, g1, be1, b2, g2, be2, b3, g3, be3 = vecs
    return pl.pallas_call(
        _edge_mlp_body,
        grid=(Ec // be,),
        in_specs=[pl.BlockSpec(memory_space=pl.ANY),
                  bs_e, bs_e, bs_full,
                  bs_v, bs_v, bs_v,
                  bs_w, bs_v, bs_v, bs_v,
                  bs_w, bs_v, bs_v, bs_v],
        out_specs=bs_full,
        out_shape=jax.ShapeDtypeStruct((E, _H), jnp.float32),
        input_output_aliases={0: 0},
    )(buf, ga, gb, te, b1, g1, be1, w2, b2, g2, be2, w3, b3, g3, be3)


def _row(v):
    return v.reshape(1, _H)


def kernel(nodes, edges, graph, nodes_res, edges_res, params):
    N = nodes.shape[0]
    E = edges.shape[0]
    npar = params["node_network"]
    epar = params["edge_network"]
    w1n, w2n, w3n = npar["W"]
    w1e, w2e, w3e = epar["W"]

    t_node, t_edge = _edge_pre(
        edges, edges_res, w1n[2 * _H:], w1e[2 * _H:], be=8000
    )

    src2 = graph[0].reshape(E // 128, 128)
    dst2 = graph[1].reshape(E // 128, 128)

    NP = (N + 127) // 128 * 128
    partials = _make_scatter(E, NP)(t_node, src2, dst2)
    partials = partials[:, :N]

    nvecs = (_row(npar["b"][0]), _row(npar["g"][0]), _row(npar["beta"][0]),
             _row(npar["b"][1]), _row(npar["g"][1]), _row(npar["beta"][1]),
             _row(npar["b"][2]), _row(npar["g"][2]), _row(npar["beta"][2]))
    new_nodes, a, b = _node_mlp(
        nodes, nodes_res, partials[0], partials[1],
        w1n[: 2 * _H], (w2n, w3n), nvecs,
        w1e[:_H], w1e[_H: 2 * _H], bn=2000,
    )

    evecs = (_row(epar["b"][0]), _row(epar["g"][0]), _row(epar["beta"][0]),
             _row(epar["b"][1]), _row(epar["g"][1]), _row(epar["beta"][1]),
             _row(epar["b"][2]), _row(epar["g"][2]), _row(epar["beta"][2]))
    CH = 4
    BE = 8000
    Ec = E // CH
    rows_c = (E // 128) // CH
    gather_k = _make_gather(Ec, N)
    gs = []
    for i in range(CH):
        s2 = lax.slice_in_dim(src2, i * rows_c, (i + 1) * rows_c)
        d2 = lax.slice_in_dim(dst2, i * rows_c, (i + 1) * rows_c)
        gs.append(gather_k(a, b, s2, d2))
    buf = t_node
    for i in range(CH):
        ga, gb = gs[i]
        buf = _edge_mlp(buf, ga, gb, t_edge, i * (Ec // BE), (w2e, w3e),
                        evecs, be=BE)
    new_edges = buf

    return (new_nodes, new_edges)


# speedup vs baseline: 6.0759x; 1.0386x over previous
import functools

import jax
import jax.numpy as jnp
from jax import lax
from jax.experimental import pallas as pl
from jax.experimental.pallas import tpu as pltpu
from jax.experimental.pallas import tpu_sc as plsc

_H = 128
_EPS = 1e-5
_PREC = None


def _ln(x, g, b):
    mu = jnp.mean(x, axis=-1, keepdims=True)
    d = x - mu
    var = jnp.mean(d * d, axis=-1, keepdims=True)
    return d * lax.rsqrt(var + _EPS) * g + b


def _tanh(x):
    return 1.0 - 2.0 / (jnp.exp(2.0 * x) + 1.0)


def _dot(x, w):
    return jnp.dot(x, w, preferred_element_type=jnp.float32, precision=_PREC)


def _edge_pre_body(e_ref, er_ref, wn_ref, we_ref, tn_ref, te_ref):
    x = jnp.concatenate([e_ref[...], er_ref[...]], axis=-1)
    tn_ref[...] = _dot(x, wn_ref[...])
    te_ref[...] = _dot(x, we_ref[...]).astype(jnp.bfloat16)


def _edge_pre(edges, edges_res, wn, we, be, block0, nblk):
    bs_in = pl.BlockSpec((be, _H), lambda i: (block0 + i, 0))
    bs_out = pl.BlockSpec((be, _H), lambda i: (i, 0))
    bs_w = pl.BlockSpec((2 * _H, _H), lambda i: (0, 0))
    return pl.pallas_call(
        _edge_pre_body,
        grid=(nblk,),
        in_specs=[bs_in, bs_in, bs_w, bs_w],
        out_specs=[bs_out, bs_out],
        out_shape=(
            jax.ShapeDtypeStruct((nblk * be, _H), jnp.float32),
            jax.ShapeDtypeStruct((nblk * be, _H), jnp.bfloat16),
        ),
    )(edges, edges_res, wn, we)


def _make_scatter(E, NP):
    mesh = plsc.VectorSubcoreMesh(core_axis_name="c", subcore_axis_name="s")
    nchunks = E // 128
    rows_per_sub = NP // 16
    zrows = 128

    @functools.partial(
        pl.kernel,
        out_type=jax.ShapeDtypeStruct((2, NP, _H), jnp.float32),
        mesh=mesh,
        scratch_types=[
            pltpu.VMEM_SHARED((NP, _H), jnp.float32),
            pltpu.VMEM((zrows, _H), jnp.float32),
            pltpu.SemaphoreType.DMA,
            pltpu.SemaphoreType.DMA,
        ],
    )
    def k(t_hbm, src_hbm, dst_hbm, out_hbm, acc, zbuf, sem1, sem2):
        c = lax.axis_index("c")
        s = lax.axis_index("s")

        @pl.loop(0, zrows)
        def _zero_rows(r):
            @pl.loop(0, _H, step=16)
            def _zero_cols(col):
                zbuf[r, pl.ds(col, 16)] = jnp.zeros((16,), jnp.float32)

        @pl.loop(0, rows_per_sub, step=zrows)
        def _fill(r0):
            pltpu.sync_copy(zbuf, acc.at[pl.ds(s * rows_per_sub + r0, zrows)])

        plsc.subcore_barrier()

        def body(t_vmem, s_vmem, d_vmem):
            c1 = pltpu.async_copy(t_vmem, acc.at[s_vmem.at[0]], sem1, add=True)
            c2 = pltpu.async_copy(t_vmem, acc.at[d_vmem.at[0]], sem2, add=True)
            c1.wait()
            c2.wait()

        pltpu.emit_pipeline(
            body,
            grid=(nchunks,),
            in_specs=[
                pl.BlockSpec((128, _H), lambda i: (i, 0)),
                pl.BlockSpec((1, 128), lambda i: (i, 0)),
                pl.BlockSpec((1, 128), lambda i: (i, 0)),
            ],
            core_axis_name=("c", "s"),
            dimension_semantics=(pltpu.PARALLEL,),
        )(t_hbm, src_hbm, dst_hbm)

        plsc.subcore_barrier()
        pltpu.sync_copy(
            acc.at[pl.ds(s * rows_per_sub, rows_per_sub)],
            out_hbm.at[c, pl.ds(s * rows_per_sub, rows_per_sub)],
        )

    return k


def _node_mlp_body(
    n_ref, nr_ref, m0_ref, m1_ref,
    w1_ref, b1_ref, g1_ref, be1_ref,
    w2_ref, b2_ref, g2_ref, be2_ref,
    w3_ref, b3_ref, g3_ref, be3_ref,
    wsrc_ref, wdst_ref,
    nn_ref, a_ref, b_ref,
):
    x = jnp.concatenate([n_ref[...], nr_ref[...]], axis=-1)
    msg = (m0_ref[0] + m0_ref[1]) + (m1_ref[0] + m1_ref[1])
    h = _dot(x, w1_ref[...]) + msg + b1_ref[...]
    h = jax.nn.relu(_ln(h, g1_ref[...], be1_ref[...]))
    h = _dot(h, w2_ref[...]) + b2_ref[...]
    h = jax.nn.relu(_ln(h, g2_ref[...], be2_ref[...]))
    h = _dot(h, w3_ref[...]) + b3_ref[...]
    h = _tanh(_ln(h, g3_ref[...], be3_ref[...]))
    nn_ref[...] = h
    a_ref[...] = _dot(h, wsrc_ref[...])
    b_ref[...] = _dot(h, wdst_ref[...])


def _node_mlp(nodes, nodes_res, m0, m1, w1, mats, vecs, wsrc, wdst, bn):
    N = nodes.shape[0]
    bs_n = pl.BlockSpec((bn, _H), lambda i: (i, 0))
    bs_m = pl.BlockSpec((2, bn, _H), lambda i: (0, i, 0))
    bs_w1 = pl.BlockSpec((2 * _H, _H), lambda i: (0, 0))
    bs_w = pl.BlockSpec((_H, _H), lambda i: (0, 0))
    bs_v = pl.BlockSpec((1, _H), lambda i: (0, 0))
    w2, w3 = mats
    b1, g1, be1, b2, g2, be2, b3, g3, be3 = vecs
    return pl.pallas_call(
        _node_mlp_body,
        grid=(N // bn,),
        in_specs=[bs_n, bs_n, bs_m, bs_m,
                  bs_w1, bs_v, bs_v, bs_v,
                  bs_w, bs_v, bs_v, bs_v,
                  bs_w, bs_v, bs_v, bs_v,
                  bs_w, bs_w],
        out_specs=[bs_n, bs_n, bs_n],
        out_shape=(
            jax.ShapeDtypeStruct((N, _H), jnp.float32),
            jax.ShapeDtypeStruct((N, _H), jnp.float32),
            jax.ShapeDtypeStruct((N, _H), jnp.float32),
        ),
    )(nodes, nodes_res, m0, m1,
      w1, b1, g1, be1, w2, b2, g2, be2, w3, b3, g3, be3, wsrc, wdst)


def _make_gather(E, N):
    mesh = plsc.VectorSubcoreMesh(core_axis_name="c", subcore_axis_name="s")
    nchunks = E // 128

    @functools.partial(
        pl.kernel,
        out_type=(
            jax.ShapeDtypeStruct((E, _H), jnp.float32),
            jax.ShapeDtypeStruct((E, _H), jnp.float32),
        ),
        mesh=mesh,
    )
    def k(a_hbm, b_hbm, src_hbm, dst_hbm, ga_hbm, gb_hbm):
        def body(s_vmem, d_vmem, ga_vmem, gb_vmem):
            pltpu.sync_copy(a_hbm.at[s_vmem.at[0]], ga_vmem)
            pltpu.sync_copy(b_hbm.at[d_vmem.at[0]], gb_vmem)

        pltpu.emit_pipeline(
            body,
            grid=(nchunks,),
            in_specs=[
                pl.BlockSpec((1, 128), lambda i: (i, 0)),
                pl.BlockSpec((1, 128), lambda i: (i, 0)),
            ],
            out_specs=[
                pl.BlockSpec((128, _H), lambda i: (i, 0)),
                pl.BlockSpec((128, _H), lambda i: (i, 0)),
            ],
            core_axis_name=("c", "s"),
            dimension_semantics=(pltpu.PARALLEL,),
        )(src_hbm, dst_hbm, ga_hbm, gb_hbm)

    return k


def _edge_mlp_body(
    buf_ref, ga_ref, gb_ref, te_ref,
    b1_ref, g1_ref, be1_ref,
    w2_ref, b2_ref, g2_ref, be2_ref,
    w3_ref, b3_ref, g3_ref, be3_ref,
    out_ref,
):
    h = (ga_ref[...].astype(jnp.float32) + gb_ref[...].astype(jnp.float32)
         + te_ref[...].astype(jnp.float32) + b1_ref[...])
    h = jax.nn.relu(_ln(h, g1_ref[...], be1_ref[...]))
    h = _dot(h, w2_ref[...]) + b2_ref[...]
    h = jax.nn.relu(_ln(h, g2_ref[...], be2_ref[...]))
    h = _dot(h, w3_ref[...]) + b3_ref[...]
    out_ref[...] = _tanh(_ln(h, g3_ref[...], be3_ref[...]))


def _edge_mlp(buf, E, ga, gb, te, te_block0, out_block0, mats, vecs, be):
    Ec = ga.shape[0]
    bs_e = pl.BlockSpec((be, _H), lambda i: (i, 0))
    bs_te = pl.BlockSpec((be, _H), lambda i: (te_block0 + i, 0))
    bs_out = pl.BlockSpec((be, _H), lambda i: (out_block0 + i, 0))
    bs_w = pl.BlockSpec((_H, _H), lambda i: (0, 0))
    bs_v = pl.BlockSpec((1, _H), lambda i: (0, 0))
    w2, w3 = mats
    b1, g1, be1, b2, g2, be2, b3, g3, be3 = vecs
    in_specs = [bs_e, bs_e, bs_te,
                bs_v, bs_v, bs_v,
                bs_w, bs_v, bs_v, bs_v,
                bs_w, bs_v, bs_v, bs_v]
    args = [ga, gb, te, b1, g1, be1, w2, b2, g2, be2, w3, b3, g3, be3]
    kwargs = {}
    body = _edge_mlp_body
    if buf is not None:
        in_specs = [pl.BlockSpec(memory_space=pl.ANY)] + in_specs
        args = [buf] + args
        kwargs["input_output_aliases"] = {0: 0}
    else:
        def body(*refs):
            return _edge_mlp_body(None, *refs)
    return pl.pallas_call(
        body,
        grid=(Ec // be,),
        in_specs=in_specs,
        out_specs=bs_out,
        out_shape=jax.ShapeDtypeStruct((E, _H), jnp.float32),
        **kwargs,
    )(*args)


def _row(v):
    return v.reshape(1, _H)


def kernel(nodes, edges, graph, nodes_res, edges_res, params):
    N = nodes.shape[0]
    E = edges.shape[0]
    npar = params["node_network"]
    epar = params["edge_network"]
    w1n, w2n, w3n = npar["W"]
    w1e, w2e, w3e = epar["W"]

    src2 = graph[0].reshape(E // 128, 128)
    dst2 = graph[1].reshape(E // 128, 128)
    NP = (N + 127) // 128 * 128

    BE1 = 8000
    Eh = E // 2
    rows_h = (E // 128) // 2
    scatter_k = _make_scatter(Eh, NP)
    te_halves = []
    partials = []
    for h in range(2):
        tn_h, te_h = _edge_pre(
            edges, edges_res, w1n[2 * _H:], w1e[2 * _H:], be=BE1,
            block0=h * (Eh // BE1), nblk=Eh // BE1,
        )
        s2 = lax.slice_in_dim(src2, h * rows_h, (h + 1) * rows_h)
        d2 = lax.slice_in_dim(dst2, h * rows_h, (h + 1) * rows_h)
        partials.append(scatter_k(tn_h, s2, d2))
        te_halves.append(te_h)

    nvecs = (_row(npar["b"][0]), _row(npar["g"][0]), _row(npar["beta"][0]),
             _row(npar["b"][1]), _row(npar["g"][1]), _row(npar["beta"][1]),
             _row(npar["b"][2]), _row(npar["g"][2]), _row(npar["beta"][2]))
    new_nodes, a, b = _node_mlp(
        nodes, nodes_res, partials[0], partials[1],
        w1n[: 2 * _H], (w2n, w3n), nvecs,
        w1e[:_H], w1e[_H: 2 * _H], bn=2000,
    )
    del partials

    evecs = (_row(epar["b"][0]), _row(epar["g"][0]), _row(epar["beta"][0]),
             _row(epar["b"][1]), _row(epar["g"][1]), _row(epar["beta"][1]),
             _row(epar["b"][2]), _row(epar["g"][2]), _row(epar["beta"][2]))
    CH = 4
    BE = 8000
    Ec = E // CH
    rows_c = (E // 128) // CH
    gather_k = _make_gather(Ec, N)
    gs = []
    for i in range(CH):
        s2 = lax.slice_in_dim(src2, i * rows_c, (i + 1) * rows_c)
        d2 = lax.slice_in_dim(dst2, i * rows_c, (i + 1) * rows_c)
        gs.append(gather_k(a, b, s2, d2))
    buf = None
    for i in range(CH):
        ga, gb = gs[i]
        buf = _edge_mlp(buf, E, ga, gb, te_halves[i // 2],
                        (i % 2) * (Ec // BE), i * (Ec // BE),
                        (w2e, w3e), evecs, be=BE)
    new_edges = buf

    return (new_nodes, new_edges)
